# Initial kernel scaffold; baseline (speedup 1.0000x reference)
import jax
import jax.numpy as jnp
from jax.experimental import pallas as pl


def kernel(x, router_w, router_b, w1, b1, w2, b2):
    raise NotImplementedError("write your pallas kernel here")


# speedup vs baseline: 1.0255x; 1.0255x over previous
import functools

import jax
import jax.numpy as jnp
from jax.experimental import pallas as pl
from jax.experimental.pallas import tpu as pltpu

B, S, D, E, F = 32, 4, 1024, 8, 2048
T = B * S
FB = 512
NF = F // FB


def _moe_kernel(x_ref, rw_ref, rb_ref, w1_ref, b1_ref, w2_ref, b2_ref,
                out_ref, loss_ref, logits_ref, acc_ref, scale_ref):
    e = pl.program_id(0)
    f = pl.program_id(1)

    @pl.when((e == 0) & (f == 0))
    def _router():
        xx = x_ref[...]
        logits = jnp.dot(xx, rw_ref[...],
                         preferred_element_type=jnp.float32) + rb_ref[...]
        logits_ref[...] = logits
        m = jnp.max(logits, axis=-1, keepdims=True)
        ex = jnp.exp(logits - m)
        probs = ex / jnp.sum(ex, axis=-1, keepdims=True)
        pmax = jnp.max(probs, axis=-1, keepdims=True)
        lane = jax.lax.broadcasted_iota(jnp.int32, (T, E), 1)
        first = jnp.min(jnp.where(probs == pmax, lane, E), axis=-1,
                        keepdims=True)
        onehot = (lane == first).astype(jnp.float32)
        scale_ref[...] = onehot * pmax
        f_frac = jnp.sum(onehot, axis=0) * (1.0 / T)
        p_mean = jnp.sum(probs, axis=0) * (1.0 / T)
        loss_ref[...] = (E * jnp.sum(f_frac * p_mean)).reshape(1, 1)

    xx = x_ref[...].astype(jnp.bfloat16)
    h = jnp.dot(xx, w1_ref[0].astype(jnp.bfloat16),
                preferred_element_type=jnp.float32)
    h = jnp.maximum(h + b1_ref[0], 0.0).astype(jnp.bfloat16)
    part = jnp.dot(h, w2_ref[0].astype(jnp.bfloat16),
                   preferred_element_type=jnp.float32)

    @pl.when(f == 0)
    def _init():
        acc_ref[...] = part

    @pl.when(f > 0)
    def _acc():
        acc_ref[...] += part

    @pl.when(f == NF - 1)
    def _emit():
        lane = jax.lax.broadcasted_iota(jnp.int32, (T, E), 1)
        s = jnp.sum(scale_ref[...] * (lane == e).astype(jnp.float32),
                    axis=1, keepdims=True)
        contrib = s * (acc_ref[...] + b2_ref[0])

        @pl.when(e == 0)
        def _first():
            out_ref[...] = contrib

        @pl.when(e > 0)
        def _rest():
            out_ref[...] += contrib


@functools.partial(jax.jit, static_argnames=("interpret",))
def _moe(x, router_w, router_b, w1, b1, w2, b2, interpret=False):
    x_flat = x.reshape(T, D)
    rb = router_b.reshape(1, E)
    b1r = b1.reshape(E, 1, F)
    b2r = b2.reshape(E, 1, D)
    grid = (E, NF)
    out, loss, logits = pl.pallas_call(
        _moe_kernel,
        grid=grid,
        in_specs=[
            pl.BlockSpec((T, D), lambda e, f: (0, 0)),
            pl.BlockSpec((D, E), lambda e, f: (0, 0)),
            pl.BlockSpec((1, E), lambda e, f: (0, 0)),
            pl.BlockSpec((1, D, FB), lambda e, f: (e, 0, f)),
            pl.BlockSpec((1, 1, FB), lambda e, f: (e, 0, f)),
            pl.BlockSpec((1, FB, D), lambda e, f: (e, f, 0)),
            pl.BlockSpec((1, 1, D), lambda e, f: (e, 0, 0)),
        ],
        out_specs=[
            pl.BlockSpec((T, D), lambda e, f: (0, 0)),
            pl.BlockSpec((1, 1), lambda e, f: (0, 0)),
            pl.BlockSpec((T, E), lambda e, f: (0, 0)),
        ],
        out_shape=[
            jax.ShapeDtypeStruct((T, D), jnp.float32),
            jax.ShapeDtypeStruct((1, 1), jnp.float32),
            jax.ShapeDtypeStruct((T, E), jnp.float32),
        ],
        scratch_shapes=[
            pltpu.VMEM((T, D), jnp.float32),
            pltpu.VMEM((T, E), jnp.float32),
        ],
        compiler_params=pltpu.CompilerParams(
            dimension_semantics=("arbitrary", "arbitrary")),
        interpret=interpret,
    )(x_flat, router_w, rb, w1, b1r, w2, b2r)
    return out.reshape(B, S, D), loss[0, 0], logits


def kernel(x, router_w, router_b, w1, b1, w2, b2):
    return _moe(x, router_w, router_b, w1, b1, w2, b2)


# speedup vs baseline: 1.1483x; 1.1197x over previous
import functools

import jax
import jax.numpy as jnp
from jax.experimental import pallas as pl
from jax.experimental.pallas import tpu as pltpu

B, S, D, E, F = 32, 4, 1024, 8, 2048
T = B * S


def _moe_kernel(x_ref, rw_ref, rb_ref, w1_ref, b1_ref, w2_ref, b2_ref,
                out_ref, loss_ref, logits_ref, scale_ref):
    e = pl.program_id(0)

    @pl.when(e == 0)
    def _router():
        xx = x_ref[...]
        logits = jnp.dot(xx, rw_ref[...],
                         preferred_element_type=jnp.float32) + rb_ref[...]
        logits_ref[...] = logits
        m = jnp.max(logits, axis=-1, keepdims=True)
        ex = jnp.exp(logits - m)
        probs = ex / jnp.sum(ex, axis=-1, keepdims=True)
        pmax = jnp.max(probs, axis=-1, keepdims=True)
        lane = jax.lax.broadcasted_iota(jnp.int32, (T, E), 1)
        first = jnp.min(jnp.where(probs == pmax, lane, E), axis=-1,
                        keepdims=True)
        onehot = (lane == first).astype(jnp.float32)
        scale_ref[...] = onehot * pmax
        f_frac = jnp.sum(onehot, axis=0) * (1.0 / T)
        p_mean = jnp.sum(probs, axis=0) * (1.0 / T)
        loss_ref[...] = (E * jnp.sum(f_frac * p_mean)).reshape(1, 1)

    xx = x_ref[...].astype(jnp.bfloat16)
    h = jnp.dot(xx, w1_ref[0].astype(jnp.bfloat16),
                preferred_element_type=jnp.float32)
    h = jnp.maximum(h + b1_ref[0], 0.0).astype(jnp.bfloat16)
    part = jnp.dot(h, w2_ref[0].astype(jnp.bfloat16),
                   preferred_element_type=jnp.float32)

    lane = jax.lax.broadcasted_iota(jnp.int32, (T, E), 1)
    s = jnp.sum(scale_ref[...] * (lane == e).astype(jnp.float32),
                axis=1, keepdims=True)
    contrib = s * (part + b2_ref[0])

    @pl.when(e == 0)
    def _first():
        out_ref[...] = contrib

    @pl.when(e > 0)
    def _rest():
        out_ref[...] += contrib


@functools.partial(jax.jit, static_argnames=("interpret",))
def _moe(x, router_w, router_b, w1, b1, w2, b2, interpret=False):
    x_flat = x.reshape(T, D)
    rb = router_b.reshape(1, E)
    b1r = b1.reshape(E, 1, F)
    b2r = b2.reshape(E, 1, D)
    out, loss, logits = pl.pallas_call(
        _moe_kernel,
        grid=(E,),
        in_specs=[
            pl.BlockSpec((T, D), lambda e: (0, 0)),
            pl.BlockSpec((D, E), lambda e: (0, 0)),
            pl.BlockSpec((1, E), lambda e: (0, 0)),
            pl.BlockSpec((1, D, F), lambda e: (e, 0, 0)),
            pl.BlockSpec((1, 1, F), lambda e: (e, 0, 0)),
            pl.BlockSpec((1, F, D), lambda e: (e, 0, 0)),
            pl.BlockSpec((1, 1, D), lambda e: (e, 0, 0)),
        ],
        out_specs=[
            pl.BlockSpec((T, D), lambda e: (0, 0)),
            pl.BlockSpec((1, 1), lambda e: (0, 0)),
            pl.BlockSpec((T, E), lambda e: (0, 0)),
        ],
        out_shape=[
            jax.ShapeDtypeStruct((T, D), jnp.float32),
            jax.ShapeDtypeStruct((1, 1), jnp.float32),
            jax.ShapeDtypeStruct((T, E), jnp.float32),
        ],
        scratch_shapes=[
            pltpu.VMEM((T, E), jnp.float32),
        ],
        compiler_params=pltpu.CompilerParams(
            dimension_semantics=("arbitrary",)),
        interpret=interpret,
    )(x_flat, router_w, rb, w1, b1r, w2, b2r)
    return out.reshape(B, S, D), loss[0, 0], logits


def kernel(x, router_w, router_b, w1, b1, w2, b2):
    return _moe(x, router_w, router_b, w1, b1, w2, b2)


# speedup vs baseline: 1.1935x; 1.0394x over previous
import functools

import jax
import jax.numpy as jnp
from jax.experimental import pallas as pl
from jax.experimental.pallas import tpu as pltpu

B, S, D, E, F = 32, 4, 1024, 8, 2048
T = B * S


def _moe_kernel(x_ref, rw_ref, rb_ref, w1a_ref, w1b_ref, b1_ref,
                w2a_ref, w2b_ref, b2_ref,
                out_ref, loss_ref, logits_ref, scale_ref):
    e = pl.program_id(0)

    @pl.when(e == 0)
    def _router():
        xx = x_ref[...]
        logits = jnp.dot(xx, rw_ref[...],
                         preferred_element_type=jnp.float32) + rb_ref[...]
        logits_ref[...] = logits
        m = jnp.max(logits, axis=-1, keepdims=True)
        ex = jnp.exp(logits - m)
        probs = ex / jnp.sum(ex, axis=-1, keepdims=True)
        pmax = jnp.max(probs, axis=-1, keepdims=True)
        lane = jax.lax.broadcasted_iota(jnp.int32, (T, E), 1)
        first = jnp.min(jnp.where(probs == pmax, lane, E), axis=-1,
                        keepdims=True)
        onehot = (lane == first).astype(jnp.float32)
        scale_ref[...] = onehot * pmax
        f_frac = jnp.sum(onehot, axis=0) * (1.0 / T)
        p_mean = jnp.sum(probs, axis=0) * (1.0 / T)
        loss_ref[...] = (E * jnp.sum(f_frac * p_mean)).reshape(1, 1)

    xx = x_ref[...].astype(jnp.bfloat16)
    h = (jnp.dot(xx[:, :D // 2], w1a_ref[0].astype(jnp.bfloat16),
                 preferred_element_type=jnp.float32) +
         jnp.dot(xx[:, D // 2:], w1b_ref[0].astype(jnp.bfloat16),
                 preferred_element_type=jnp.float32))
    h = jnp.maximum(h + b1_ref[0], 0.0).astype(jnp.bfloat16)
    part = (jnp.dot(h[:, :F // 2], w2a_ref[0].astype(jnp.bfloat16),
                    preferred_element_type=jnp.float32) +
            jnp.dot(h[:, F // 2:], w2b_ref[0].astype(jnp.bfloat16),
                    preferred_element_type=jnp.float32))

    lane = jax.lax.broadcasted_iota(jnp.int32, (T, E), 1)
    s = jnp.sum(scale_ref[...] * (lane == e).astype(jnp.float32),
                axis=1, keepdims=True)
    contrib = s * (part + b2_ref[0])

    @pl.when(e == 0)
    def _first():
        out_ref[...] = contrib

    @pl.when(e > 0)
    def _rest():
        out_ref[...] += contrib


@functools.partial(jax.jit, static_argnames=("interpret",))
def _moe(x, router_w, router_b, w1, b1, w2, b2, interpret=False):
    x_flat = x.reshape(T, D)
    rb = router_b.reshape(1, E)
    b1r = b1.reshape(E, 1, F)
    b2r = b2.reshape(E, 1, D)
    out, loss, logits = pl.pallas_call(
        _moe_kernel,
        grid=(E,),
        in_specs=[
            pl.BlockSpec((T, D), lambda e: (0, 0)),
            pl.BlockSpec((D, E), lambda e: (0, 0)),
            pl.BlockSpec((1, E), lambda e: (0, 0)),
            pl.BlockSpec((1, D // 2, F), lambda e: (e, 0, 0)),
            pl.BlockSpec((1, D // 2, F), lambda e: (e, 1, 0)),
            pl.BlockSpec((1, 1, F), lambda e: (e, 0, 0)),
            pl.BlockSpec((1, F // 2, D), lambda e: (e, 0, 0)),
            pl.BlockSpec((1, F // 2, D), lambda e: (e, 1, 0)),
            pl.BlockSpec((1, 1, D), lambda e: (e, 0, 0)),
        ],
        out_specs=[
            pl.BlockSpec((T, D), lambda e: (0, 0)),
            pl.BlockSpec((1, 1), lambda e: (0, 0)),
            pl.BlockSpec((T, E), lambda e: (0, 0)),
        ],
        out_shape=[
            jax.ShapeDtypeStruct((T, D), jnp.float32),
            jax.ShapeDtypeStruct((1, 1), jnp.float32),
            jax.ShapeDtypeStruct((T, E), jnp.float32),
        ],
        scratch_shapes=[
            pltpu.VMEM((T, E), jnp.float32),
        ],
        compiler_params=pltpu.CompilerParams(
            dimension_semantics=("arbitrary",)),
        interpret=interpret,
    )(x_flat, router_w, rb, w1, w1, b1r, w2, w2, b2r)
    return out.reshape(B, S, D), loss[0, 0], logits


def kernel(x, router_w, router_b, w1, b1, w2, b2):
    return _moe(x, router_w, router_b, w1, b1, w2, b2)
